# Initial kernel scaffold; baseline (speedup 1.0000x reference)
import jax
import jax.numpy as jnp
from jax import lax
from jax.experimental import pallas as pl
from jax.experimental.pallas import tpu as pltpu
from jax.experimental.pallas import tpu_sc as plsc

VOCAB = 1000000
EMB_DIM = 64
BATCH = 16384
HIST = 50

NUM_CORES = 2
NUM_SUBCORES = 16
NUM_WORKERS = NUM_CORES * NUM_SUBCORES

B_TOTAL = BATCH * HIST
B_PER_W = B_TOTAL // NUM_WORKERS

IDX_W = 128
K = 10
CHUNK = K * IDX_W
N_CHUNKS = B_PER_W // CHUNK


def _body(idx_hbm, table_hbm, out_hbm, idx_v, rows_v, gsem):
    wid = lax.axis_index("s") * NUM_CORES + lax.axis_index("c")
    base_row = wid * (B_PER_W // IDX_W)

    def chunk_body(i, carry):
        row_off = base_row + i * K
        pltpu.sync_copy(idx_hbm.at[pl.ds(row_off, K)], idx_v)
        copies = [
            pltpu.async_copy(
                table_hbm.at[idx_v.at[j]],
                rows_v.at[pl.ds(j * IDX_W, IDX_W)],
                gsem,
            )
            for j in range(K)
        ]
        for cp in copies:
            cp.wait()
        pltpu.sync_copy(rows_v, out_hbm.at[pl.ds(row_off * IDX_W, CHUNK)])
        return carry

    lax.fori_loop(0, N_CHUNKS, chunk_body, 0)


@jax.jit
def kernel(batch, table):
    idx2 = batch.astype(jnp.int32).reshape(B_TOTAL // IDX_W, IDX_W)
    mesh = plsc.VectorSubcoreMesh(core_axis_name="c", subcore_axis_name="s")
    out_flat = pl.kernel(
        _body,
        out_type=jax.ShapeDtypeStruct((B_TOTAL, EMB_DIM), jnp.float32),
        mesh=mesh,
        scratch_types=[
            pltpu.VMEM((K, IDX_W), jnp.int32),
            pltpu.VMEM((CHUNK, EMB_DIM), jnp.float32),
            pltpu.SemaphoreType.DMA,
        ],
    )(idx2, table)
    return out_flat.reshape(BATCH, HIST, EMB_DIM)


# speedup vs baseline: 1.8447x; 1.8447x over previous
import jax
import jax.numpy as jnp
from jax import lax
from jax.experimental import pallas as pl
from jax.experimental.pallas import tpu as pltpu
from jax.experimental.pallas import tpu_sc as plsc

VOCAB = 1000000
EMB_DIM = 64
BATCH = 16384
HIST = 50

NUM_CORES = 2
NUM_SUBCORES = 16
NUM_WORKERS = NUM_CORES * NUM_SUBCORES

B_TOTAL = BATCH * HIST
B_PER_W = B_TOTAL // NUM_WORKERS

IDX_W = 128
K = 8
CHUNK = K * IDX_W
N_CHUNKS = B_PER_W // CHUNK


def _body(idx_hbm, table_hbm, out_hbm, idx_v, rows_v, gsem):
    wid = lax.axis_index("s") * NUM_CORES + lax.axis_index("c")
    base_row = wid * (B_PER_W // IDX_W)

    def chunk_body(i, carry):
        row_off = base_row + i * K
        pltpu.sync_copy(idx_hbm.at[pl.ds(row_off, K)], idx_v)
        copies = [
            pltpu.async_copy(
                table_hbm.at[idx_v.at[j]],
                rows_v.at[pl.ds(j * IDX_W, IDX_W)],
                gsem,
            )
            for j in range(K)
        ]
        for cp in copies:
            cp.wait()
        pltpu.sync_copy(rows_v, out_hbm.at[pl.ds(row_off * IDX_W, CHUNK)])
        return carry

    lax.fori_loop(0, N_CHUNKS, chunk_body, 0)


@jax.jit
def kernel(batch, table):
    idx2 = batch.astype(jnp.int32).reshape(B_TOTAL // IDX_W, IDX_W)
    mesh = plsc.VectorSubcoreMesh(core_axis_name="c", subcore_axis_name="s")
    out_flat = pl.kernel(
        _body,
        out_type=jax.ShapeDtypeStruct((B_TOTAL, EMB_DIM), jnp.float32),
        mesh=mesh,
        compiler_params=pltpu.CompilerParams(use_tc_tiling_on_sc=False),
        scratch_types=[
            pltpu.VMEM((K, IDX_W), jnp.int32),
            pltpu.VMEM((CHUNK, EMB_DIM), jnp.float32),
            pltpu.SemaphoreType.DMA,
        ],
    )(idx2, table)
    return out_flat.reshape(BATCH, HIST, EMB_DIM)
